# Initial kernel scaffold; baseline (speedup 1.0000x reference)
import jax
import jax.numpy as jnp
from jax.experimental import pallas as pl


def kernel(x, pos, norm, W0, b0, W1, b1, Wq, bq, Wk, bk, Wv, bv, We, be, Wr, br, Wout, bout, edge_index):
    raise NotImplementedError("write your pallas kernel here")


# speedup vs baseline: 3.4649x; 3.4649x over previous
import functools

import jax
import jax.numpy as jnp
from jax.experimental import pallas as pl
from jax.experimental.pallas import tpu as pltpu

_N = 10000
_E = 320000
_NHID = 16
_DEPTH = 3

_BN = 2000
_BE = 8000


def _lin_in_body(x_ref, w0_ref, b0_ref, w1_ref, b1_ref, o_ref):
    h = jnp.maximum(x_ref[...] @ w0_ref[...] + b0_ref[...], 0.0)
    o_ref[...] = jnp.maximum(h @ w1_ref[...] + b1_ref[...], 0.0)


def _lin_in(x, W0, b0, W1, b1):
    grid = _N // _BN
    return pl.pallas_call(
        _lin_in_body,
        grid=(grid,),
        in_specs=[
            pl.BlockSpec((_BN, 128), lambda i: (i, 0)),
            pl.BlockSpec((128, 128), lambda i: (0, 0)),
            pl.BlockSpec((1, 128), lambda i: (0, 0)),
            pl.BlockSpec((128, _NHID), lambda i: (0, 0)),
            pl.BlockSpec((1, _NHID), lambda i: (0, 0)),
        ],
        out_specs=pl.BlockSpec((_BN, _NHID), lambda i: (i, 0)),
        out_shape=jax.ShapeDtypeStruct((_N, _NHID), jnp.float32),
    )(x, W0, b0.reshape(1, 128), W1, b1.reshape(1, _NHID))


def _edge_body(hs_ref, hd_ref, wq_ref, bq_ref, wka_ref, wvv_ref, bvbe_ref,
               out_ref):
    hs = hs_ref[...]
    hd = hd_ref[...]
    q = (hd @ wq_ref[...] + bq_ref[...]) * 0.25
    a = hs @ wka_ref[...]
    v = hs @ wvv_ref[...] + bvbe_ref[...]
    logit = jnp.sum(q * a, axis=-1)
    p = jnp.exp(jnp.minimum(logit, 60.0))
    msg = p[:, None] * v
    pb = jnp.broadcast_to(p[:, None], (_BE, _NHID))
    out_ref[...] = jnp.concatenate([msg, pb], axis=1)


def _edge_stage(hs, hd, Wq, bq, WkA, WvV, bvbe):
    grid = _E // _BE
    return pl.pallas_call(
        _edge_body,
        grid=(grid,),
        in_specs=[
            pl.BlockSpec((_BE, _NHID), lambda i: (i, 0)),
            pl.BlockSpec((_BE, _NHID), lambda i: (i, 0)),
            pl.BlockSpec((_NHID, _NHID), lambda i: (0, 0)),
            pl.BlockSpec((1, _NHID), lambda i: (0, 0)),
            pl.BlockSpec((_NHID, _NHID), lambda i: (0, 0)),
            pl.BlockSpec((_NHID, _NHID), lambda i: (0, 0)),
            pl.BlockSpec((1, _NHID), lambda i: (0, 0)),
        ],
        out_specs=pl.BlockSpec((_BE, 2 * _NHID), lambda i: (i, 0)),
        out_shape=jax.ShapeDtypeStruct((_E, 2 * _NHID), jnp.float32),
    )(hs, hd, Wq, bq.reshape(1, _NHID), WkA, WvV, bvbe.reshape(1, _NHID))


def _combine_body(ud_ref, h_ref, wb_ref, wr_ref, br_ref, o_ref):
    ud = jnp.sum(ud_ref[...], axis=0)
    u = ud[:, :_NHID]
    d = ud[:, _NHID]
    h = h_ref[...]
    b = h @ wb_ref[...]
    hr = h @ wr_ref[...] + br_ref[...]
    safe = d > 0.0
    dn = jnp.where(safe, d, 1.0)
    agg = jnp.where(safe[:, None], (u + b * d[:, None]) / dn[:, None], 0.0)
    o_ref[...] = jnp.maximum(agg + hr, 0.0)


def _combine_stage(UD, h, WB, Wr, br):
    grid = _N // _BN
    nu = UD.shape[0]
    return pl.pallas_call(
        _combine_body,
        grid=(grid,),
        in_specs=[
            pl.BlockSpec((nu, _BN, 2 * _NHID), lambda i: (0, i, 0)),
            pl.BlockSpec((_BN, _NHID), lambda i: (i, 0)),
            pl.BlockSpec((_NHID, _NHID), lambda i: (0, 0)),
            pl.BlockSpec((_NHID, _NHID), lambda i: (0, 0)),
            pl.BlockSpec((1, _NHID), lambda i: (0, 0)),
        ],
        out_specs=pl.BlockSpec((_BN, _NHID), lambda i: (i, 0)),
        out_shape=jax.ShapeDtypeStruct((_N, _NHID), jnp.float32),
    )(UD, h, WB, Wr, br.reshape(1, _NHID))


def _proj_body(h_ref, w_ref, b_ref, o_ref):
    o_ref[...] = h_ref[...] @ w_ref[...] + b_ref[...]


def _proj_out(h, Wout, bout):
    grid = _N // _BN
    return pl.pallas_call(
        _proj_body,
        grid=(grid,),
        in_specs=[
            pl.BlockSpec((_BN, _NHID), lambda i: (i, 0)),
            pl.BlockSpec((_NHID, 2), lambda i: (0, 0)),
            pl.BlockSpec((1, 2), lambda i: (0, 0)),
        ],
        out_specs=pl.BlockSpec((_BN, 2), lambda i: (i, 0)),
        out_shape=jax.ShapeDtypeStruct((_N, 2), jnp.float32),
    )(h, Wout, bout.reshape(1, 2))


def kernel(x, pos, norm, W0, b0, W1, b1, Wq, bq, Wk, bk, Wv, bv, We, be,
           Wr, br, Wout, bout, edge_index):
    src = edge_index[0]
    dst = edge_index[1]
    h = _lin_in(x, W0, b0, W1, b1)
    for l in range(_DEPTH):
        WkA = Wk[l] + We[l][:_NHID]
        WvV = Wv[l] + We[l][:_NHID]
        bvbe = bv[l] + be[l]
        WB = We[l][_NHID:]
        hs = jnp.take(h, src, axis=0)
        hd = jnp.take(h, dst, axis=0)
        msgp = _edge_stage(hs, hd, Wq[l], bq[l], WkA, WvV, bvbe)
        UD = jax.ops.segment_sum(msgp, dst, num_segments=_N)[None]
        h = _combine_stage(UD, h, WB, Wr[l], br[l])
    return _proj_out(h, Wout, bout)


# speedup vs baseline: 11.6884x; 3.3734x over previous
import functools

import jax
import jax.numpy as jnp
from jax import lax
from jax.experimental import pallas as pl
from jax.experimental.pallas import tpu as pltpu
from jax.experimental.pallas import tpu_sc as plsc

_N = 10000
_E = 320000
_NHID = 16
_DEPTH = 3

_NW = 32
_KC = 100
_EPW = _E // _NW
_NCH = _EPW // _KC
_NP = 10240

_BN = 2000
_BE = 8000


def _lin_in_body(x_ref, w0_ref, b0_ref, w1_ref, b1_ref, o_ref):
    h = jnp.maximum(x_ref[...] @ w0_ref[...] + b0_ref[...], 0.0)
    o_ref[...] = jnp.maximum(h @ w1_ref[...] + b1_ref[...], 0.0)


def _lin_in(x, W0, b0, W1, b1):
    grid = _N // _BN
    return pl.pallas_call(
        _lin_in_body,
        grid=(grid,),
        in_specs=[
            pl.BlockSpec((_BN, 128), lambda i: (i, 0)),
            pl.BlockSpec((128, 128), lambda i: (0, 0)),
            pl.BlockSpec((1, 128), lambda i: (0, 0)),
            pl.BlockSpec((128, _NHID), lambda i: (0, 0)),
            pl.BlockSpec((1, _NHID), lambda i: (0, 0)),
        ],
        out_specs=pl.BlockSpec((_BN, _NHID), lambda i: (i, 0)),
        out_shape=jax.ShapeDtypeStruct((_N, _NHID), jnp.float32),
    )(x, W0, b0.reshape(1, 128), W1, b1.reshape(1, _NHID))


def _edge_body(hs_ref, hd_ref, wq_ref, bq_ref, wka_ref, wvv_ref, bvbe_ref,
               out_ref):
    hs = hs_ref[...]
    hd = hd_ref[...]
    q = (hd @ wq_ref[...] + bq_ref[...]) * 0.25
    a = hs @ wka_ref[...]
    v = hs @ wvv_ref[...] + bvbe_ref[...]
    logit = jnp.sum(q * a, axis=-1)
    p = jnp.exp(jnp.minimum(logit, 60.0))
    msg = p[:, None] * v
    pb = jnp.broadcast_to(p[:, None], (_BE, _NHID))
    out_ref[...] = jnp.concatenate([msg, pb], axis=1)


def _edge_stage(hs, hd, Wq, bq, WkA, WvV, bvbe):
    grid = _E // _BE
    return pl.pallas_call(
        _edge_body,
        grid=(grid,),
        in_specs=[
            pl.BlockSpec((_BE, _NHID), lambda i: (i, 0)),
            pl.BlockSpec((_BE, _NHID), lambda i: (i, 0)),
            pl.BlockSpec((_NHID, _NHID), lambda i: (0, 0)),
            pl.BlockSpec((1, _NHID), lambda i: (0, 0)),
            pl.BlockSpec((_NHID, _NHID), lambda i: (0, 0)),
            pl.BlockSpec((_NHID, _NHID), lambda i: (0, 0)),
            pl.BlockSpec((1, _NHID), lambda i: (0, 0)),
        ],
        out_specs=pl.BlockSpec((_BE, 2 * _NHID), lambda i: (i, 0)),
        out_shape=jax.ShapeDtypeStruct((_E, 2 * _NHID), jnp.float32),
    )(hs, hd, Wq, bq.reshape(1, _NHID), WkA, WvV, bvbe.reshape(1, _NHID))


def _combine_body(ud_ref, h_ref, wb_ref, wr_ref, br_ref, o_ref):
    ud = jnp.sum(ud_ref[...], axis=0)
    u = ud[:, :_NHID]
    d = ud[:, _NHID]
    h = h_ref[...]
    b = h @ wb_ref[...]
    hr = h @ wr_ref[...] + br_ref[...]
    safe = d > 0.0
    dn = jnp.where(safe, d, 1.0)
    agg = jnp.where(safe[:, None], (u + b * d[:, None]) / dn[:, None], 0.0)
    o_ref[...] = jnp.maximum(agg + hr, 0.0)


def _combine_stage(UD, h, WB, Wr, br):
    grid = _N // _BN
    nu = UD.shape[0]
    return pl.pallas_call(
        _combine_body,
        grid=(grid,),
        in_specs=[
            pl.BlockSpec((nu, _BN, 2 * _NHID), lambda i: (0, i, 0)),
            pl.BlockSpec((_BN, _NHID), lambda i: (i, 0)),
            pl.BlockSpec((_NHID, _NHID), lambda i: (0, 0)),
            pl.BlockSpec((_NHID, _NHID), lambda i: (0, 0)),
            pl.BlockSpec((1, _NHID), lambda i: (0, 0)),
        ],
        out_specs=pl.BlockSpec((_BN, _NHID), lambda i: (i, 0)),
        out_shape=jax.ShapeDtypeStruct((_N, _NHID), jnp.float32),
    )(UD, h, WB, Wr, br.reshape(1, _NHID))


def _proj_body(h_ref, w_ref, b_ref, o_ref):
    o_ref[...] = h_ref[...] @ w_ref[...] + b_ref[...]


def _proj_out(h, Wout, bout):
    grid = _N // _BN
    return pl.pallas_call(
        _proj_body,
        grid=(grid,),
        in_specs=[
            pl.BlockSpec((_BN, _NHID), lambda i: (i, 0)),
            pl.BlockSpec((_NHID, 2), lambda i: (0, 0)),
            pl.BlockSpec((1, 2), lambda i: (0, 0)),
        ],
        out_specs=pl.BlockSpec((_BN, 2), lambda i: (i, 0)),
        out_shape=jax.ShapeDtypeStruct((_N, 2), jnp.float32),
    )(h, Wout, bout.reshape(1, 2))


def _sc_mesh():
    return plsc.VectorSubcoreMesh(core_axis_name="c", subcore_axis_name="s")


@functools.cache
def _gather_kernel():
    @functools.partial(
        pl.kernel,
        mesh=_sc_mesh(),
        out_type=[
            jax.ShapeDtypeStruct((_E // _KC, _KC, _NHID), jnp.float32),
            jax.ShapeDtypeStruct((_E // _KC, _KC, _NHID), jnp.float32),
        ],
        scratch_types=[
            pltpu.VMEM((_NCH, _KC), jnp.int32),
            pltpu.VMEM((_NCH, _KC), jnp.int32),
            pltpu.VMEM((_KC, _NHID), jnp.float32),
            pltpu.VMEM((_KC, _NHID), jnp.float32),
            pltpu.SemaphoreType.DMA,
            pltpu.SemaphoreType.DMA,
        ],
        compiler_params=pltpu.CompilerParams(use_tc_tiling_on_sc=False),
    )
    def gk(h_hbm, srcr_hbm, dstr_hbm, hs_hbm, hd_hbm,
           sidx, didx, hsbuf, hdbuf, sem0, sem1):
        wid = lax.axis_index("s") * 2 + lax.axis_index("c")
        rowbase = wid * _NCH
        pltpu.sync_copy(srcr_hbm.at[wid], sidx)
        pltpu.sync_copy(dstr_hbm.at[wid], didx)

        def body(j, carry):
            row = rowbase + j
            cps = pltpu.async_copy(h_hbm.at[sidx.at[j]], hsbuf, sem0)
            cpd = pltpu.async_copy(h_hbm.at[didx.at[j]], hdbuf, sem1)
            cps.wait()
            pltpu.sync_copy(hsbuf, hs_hbm.at[row])
            cpd.wait()
            pltpu.sync_copy(hdbuf, hd_hbm.at[row])
            return carry

        lax.fori_loop(0, _NCH, body, 0)

    return gk


@functools.cache
def _scatter_kernel():
    @functools.partial(
        pl.kernel,
        mesh=_sc_mesh(),
        out_type=jax.ShapeDtypeStruct((2, 16, _NP // 16, 2 * _NHID),
                                      jnp.float32),
        scratch_types=[
            pltpu.VMEM((_NCH, _KC), jnp.int32),
            pltpu.VMEM((_KC, 2 * _NHID), jnp.float32),
            pltpu.VMEM_SHARED((_NP, 2 * _NHID), jnp.float32),
        ],
        compiler_params=pltpu.CompilerParams(use_tc_tiling_on_sc=False),
    )
    def sk(msgp_hbm, dstr_hbm, zero_hbm, out_hbm, didx, mbuf, acc):
        c = lax.axis_index("c")
        s = lax.axis_index("s")
        wid = s * 2 + c

        @pl.when(s == 0)
        def _():
            pltpu.sync_copy(zero_hbm, acc)

        plsc.subcore_barrier()

        rowbase = wid * _NCH
        pltpu.sync_copy(dstr_hbm.at[wid], didx)

        def body(j, carry):
            row = rowbase + j
            pltpu.sync_copy(msgp_hbm.at[row], mbuf)
            pltpu.sync_copy(mbuf, acc.at[didx.at[j]], add=True)
            return carry

        lax.fori_loop(0, _NCH, body, 0)
        plsc.subcore_barrier()
        rows = _NP // 16
        pltpu.sync_copy(acc.at[pl.ds(s * rows, rows)], out_hbm.at[c, s])

    return sk


def kernel(x, pos, norm, W0, b0, W1, b1, Wq, bq, Wk, bk, Wv, bv, We, be,
           Wr, br, Wout, bout, edge_index):
    srcr = edge_index[0].reshape(_NW, _NCH, _KC)
    dstr = edge_index[1].reshape(_NW, _NCH, _KC)
    zero = jnp.zeros((_NP, 2 * _NHID), jnp.float32)
    gk = _gather_kernel()
    sk = _scatter_kernel()
    h = _lin_in(x, W0, b0, W1, b1)
    for l in range(_DEPTH):
        WkA = Wk[l] + We[l][:_NHID]
        WvV = Wv[l] + We[l][:_NHID]
        bvbe = bv[l] + be[l]
        WB = We[l][_NHID:]
        hs3, hd3 = gk(h, srcr, dstr)
        hs = hs3.reshape(_E, _NHID)
        hd = hd3.reshape(_E, _NHID)
        msgp = _edge_stage(hs, hd, Wq[l], bq[l], WkA, WvV, bvbe)
        msgp3 = msgp.reshape(_E // _KC, _KC, 2 * _NHID)
        UD4 = sk(msgp3, dstr, zero)
        UD = UD4.reshape(2, _NP, 2 * _NHID)
        h = _combine_stage(UD, h, WB, Wr[l], br[l])
    return _proj_out(h, Wout, bout)


# speedup vs baseline: 43.8846x; 3.7546x over previous
import functools

import jax
import jax.numpy as jnp
from jax import lax
from jax.experimental import pallas as pl
from jax.experimental.pallas import tpu as pltpu
from jax.experimental.pallas import tpu_sc as plsc

_N = 10000
_E = 320000
_NHID = 16
_DEPTH = 3

_NW = 32
_KC = 100
_EPW = _E // _NW
_NCH = _EPW // _KC
_NP = 10240

_BN = 2000
_BE = 8000


def _lin_in_body(x_ref, w0_ref, b0_ref, w1_ref, b1_ref, o_ref):
    h = jnp.maximum(x_ref[...] @ w0_ref[...] + b0_ref[...], 0.0)
    o_ref[...] = jnp.maximum(h @ w1_ref[...] + b1_ref[...], 0.0)


def _lin_in(x, W0, b0, W1, b1):
    grid = _N // _BN
    return pl.pallas_call(
        _lin_in_body,
        grid=(grid,),
        in_specs=[
            pl.BlockSpec((_BN, 128), lambda i: (i, 0)),
            pl.BlockSpec((128, 128), lambda i: (0, 0)),
            pl.BlockSpec((1, 128), lambda i: (0, 0)),
            pl.BlockSpec((128, _NHID), lambda i: (0, 0)),
            pl.BlockSpec((1, _NHID), lambda i: (0, 0)),
        ],
        out_specs=pl.BlockSpec((_BN, _NHID), lambda i: (i, 0)),
        out_shape=jax.ShapeDtypeStruct((_N, _NHID), jnp.float32),
    )(x, W0, b0.reshape(1, 128), W1, b1.reshape(1, _NHID))


def _combine_body(ud_ref, h_ref, wb_ref, wr_ref, br_ref, o_ref):
    ud = jnp.sum(ud_ref[...], axis=0)
    u = ud[:, :_NHID]
    d = ud[:, _NHID]
    h = h_ref[...]
    b = h @ wb_ref[...]
    hr = h @ wr_ref[...] + br_ref[...]
    safe = d > 0.0
    dn = jnp.where(safe, d, 1.0)
    agg = jnp.where(safe[:, None], (u + b * d[:, None]) / dn[:, None], 0.0)
    o_ref[...] = jnp.maximum(agg + hr, 0.0)


def _combine_stage(UD, h, WB, Wr, br):
    grid = _N // _BN
    nu = UD.shape[0]
    return pl.pallas_call(
        _combine_body,
        grid=(grid,),
        in_specs=[
            pl.BlockSpec((nu, _BN, 2 * _NHID), lambda i: (0, i, 0)),
            pl.BlockSpec((_BN, _NHID), lambda i: (i, 0)),
            pl.BlockSpec((_NHID, _NHID), lambda i: (0, 0)),
            pl.BlockSpec((_NHID, _NHID), lambda i: (0, 0)),
            pl.BlockSpec((1, _NHID), lambda i: (0, 0)),
        ],
        out_specs=pl.BlockSpec((_BN, _NHID), lambda i: (i, 0)),
        out_shape=jax.ShapeDtypeStruct((_N, _NHID), jnp.float32),
    )(UD, h, WB, Wr, br.reshape(1, _NHID))


def _proj_body(h_ref, w_ref, b_ref, o_ref):
    o_ref[...] = h_ref[...] @ w_ref[...] + b_ref[...]


def _proj_out(h, Wout, bout):
    grid = _N // _BN
    return pl.pallas_call(
        _proj_body,
        grid=(grid,),
        in_specs=[
            pl.BlockSpec((_BN, _NHID), lambda i: (i, 0)),
            pl.BlockSpec((_NHID, 2), lambda i: (0, 0)),
            pl.BlockSpec((1, 2), lambda i: (0, 0)),
        ],
        out_specs=pl.BlockSpec((_BN, 2), lambda i: (i, 0)),
        out_shape=jax.ShapeDtypeStruct((_N, 2), jnp.float32),
    )(h, Wout, bout.reshape(1, 2))


def _node_body(h_ref, wq_ref, bq_ref, wka_ref, wvv_ref, bvbe_ref,
               qt_ref, avt_ref):
    h = h_ref[...]
    qt_ref[...] = (h @ wq_ref[...] + bq_ref[...]) * 0.25
    avt_ref[...] = jnp.concatenate(
        [h @ wka_ref[...], h @ wvv_ref[...] + bvbe_ref[...]], axis=1)


def _node_stage(h, Wq, bq, WkA, WvV, bvbe):
    grid = _N // _BN
    return pl.pallas_call(
        _node_body,
        grid=(grid,),
        in_specs=[
            pl.BlockSpec((_BN, _NHID), lambda i: (i, 0)),
            pl.BlockSpec((_NHID, _NHID), lambda i: (0, 0)),
            pl.BlockSpec((1, _NHID), lambda i: (0, 0)),
            pl.BlockSpec((_NHID, _NHID), lambda i: (0, 0)),
            pl.BlockSpec((_NHID, _NHID), lambda i: (0, 0)),
            pl.BlockSpec((1, _NHID), lambda i: (0, 0)),
        ],
        out_specs=[
            pl.BlockSpec((_BN, _NHID), lambda i: (i, 0)),
            pl.BlockSpec((_BN, 2 * _NHID), lambda i: (i, 0)),
        ],
        out_shape=[
            jax.ShapeDtypeStruct((_N, _NHID), jnp.float32),
            jax.ShapeDtypeStruct((_N, 2 * _NHID), jnp.float32),
        ],
    )(h, Wq, bq.reshape(1, _NHID), WkA, WvV, bvbe.reshape(1, _NHID))


def _sc_mesh():
    return plsc.VectorSubcoreMesh(core_axis_name="c", subcore_axis_name="s")


@functools.cache
def _edge_sc_kernel():
    @functools.partial(
        pl.kernel,
        mesh=_sc_mesh(),
        out_type=jax.ShapeDtypeStruct((2, 16, _NP // 16, 2 * _NHID),
                                      jnp.float32),
        scratch_types=[
            pltpu.VMEM((_NCH, _KC), jnp.int32),
            pltpu.VMEM((_NCH, _KC), jnp.int32),
            pltpu.VMEM((_KC, _NHID), jnp.float32),
            pltpu.VMEM((_KC, 2 * _NHID), jnp.float32),
            pltpu.VMEM((_KC, 2 * _NHID), jnp.float32),
            pltpu.VMEM_SHARED((_NP, 2 * _NHID), jnp.float32),
            pltpu.SemaphoreType.DMA,
            pltpu.SemaphoreType.DMA,
        ],
        compiler_params=pltpu.CompilerParams(use_tc_tiling_on_sc=False,
                                             needs_layout_passes=False),
    )
    def ek(qt_hbm, avt_hbm, srcr_hbm, dstr_hbm, zero_hbm, out_hbm,
           sidx, didx, qbuf, avbuf, mbuf, acc, semq, sema):
        c = lax.axis_index("c")
        s = lax.axis_index("s")
        wid = s * 2 + c

        @pl.when(s == 0)
        def _():
            pltpu.sync_copy(zero_hbm, acc)

        plsc.subcore_barrier()
        pltpu.sync_copy(srcr_hbm.at[wid], sidx)
        pltpu.sync_copy(dstr_hbm.at[wid], didx)

        def chunk(j, carry):
            cpq = pltpu.async_copy(qt_hbm.at[didx.at[j]], qbuf, semq)
            cpa = pltpu.async_copy(avt_hbm.at[sidx.at[j]], avbuf, sema)
            cpq.wait()
            cpa.wait()

            @plsc.parallel_loop(0, _KC, 1, unroll=8)
            def body(i):
                q = qbuf[i, :]
                a = avbuf[i, pl.ds(0, _NHID)]
                v = avbuf[i, pl.ds(_NHID, _NHID)]
                logit = jnp.sum(q * a)
                p = jnp.exp(jnp.broadcast_to(jnp.minimum(logit, 60.0),
                                             (_NHID,)))
                mbuf[i, pl.ds(0, _NHID)] = p * v
                mbuf[i, pl.ds(_NHID, _NHID)] = p

            pltpu.sync_copy(mbuf, acc.at[didx.at[j]], add=True)
            return carry

        lax.fori_loop(0, _NCH, chunk, 0)
        plsc.subcore_barrier()
        rows = _NP // 16
        pltpu.sync_copy(acc.at[pl.ds(s * rows, rows)], out_hbm.at[c, s])

    return ek


def kernel(x, pos, norm, W0, b0, W1, b1, Wq, bq, Wk, bk, Wv, bv, We, be,
           Wr, br, Wout, bout, edge_index):
    srcr = edge_index[0].reshape(_NW, _NCH, _KC)
    dstr = edge_index[1].reshape(_NW, _NCH, _KC)
    zero = jnp.zeros((_NP, 2 * _NHID), jnp.float32)
    ek = _edge_sc_kernel()
    h = _lin_in(x, W0, b0, W1, b1)
    for l in range(_DEPTH):
        WkA = Wk[l] + We[l][:_NHID]
        WvV = Wv[l] + We[l][:_NHID]
        bvbe = bv[l] + be[l]
        WB = We[l][_NHID:]
        QT, AVT = _node_stage(h, Wq[l], bq[l], WkA, WvV, bvbe)
        UD4 = ek(QT, AVT, srcr, dstr, zero)
        UD = UD4.reshape(2, _NP, 2 * _NHID)
        h = _combine_stage(UD, h, WB, Wr[l], br[l])
    return _proj_out(h, Wout, bout)


# speedup vs baseline: 63.1286x; 1.4385x over previous
import functools

import jax
import jax.numpy as jnp
from jax import lax
from jax.experimental import pallas as pl
from jax.experimental.pallas import tpu as pltpu
from jax.experimental.pallas import tpu_sc as plsc

_N = 10000
_E = 320000
_NHID = 16
_DEPTH = 3

_NW = 32
_KC = 100
_EPW = _E // _NW
_NCH = _EPW // _KC
_NP = 10240

_BN = 2000
_BE = 8000


def _lin_in_body(x_ref, w0_ref, b0_ref, w1_ref, b1_ref, o_ref):
    h = jnp.maximum(x_ref[...] @ w0_ref[...] + b0_ref[...], 0.0)
    o_ref[...] = jnp.maximum(h @ w1_ref[...] + b1_ref[...], 0.0)


def _lin_in(x, W0, b0, W1, b1):
    grid = _N // _BN
    return pl.pallas_call(
        _lin_in_body,
        grid=(grid,),
        in_specs=[
            pl.BlockSpec((_BN, 128), lambda i: (i, 0)),
            pl.BlockSpec((128, 128), lambda i: (0, 0)),
            pl.BlockSpec((1, 128), lambda i: (0, 0)),
            pl.BlockSpec((128, _NHID), lambda i: (0, 0)),
            pl.BlockSpec((1, _NHID), lambda i: (0, 0)),
        ],
        out_specs=pl.BlockSpec((_BN, _NHID), lambda i: (i, 0)),
        out_shape=jax.ShapeDtypeStruct((_N, _NHID), jnp.float32),
    )(x, W0, b0.reshape(1, 128), W1, b1.reshape(1, _NHID))


def _combine_body(ud_ref, h_ref, wb_ref, wr_ref, br_ref, o_ref):
    ud = jnp.sum(ud_ref[...], axis=0)
    u = ud[:, :_NHID]
    d = ud[:, _NHID]
    h = h_ref[...]
    b = h @ wb_ref[...]
    hr = h @ wr_ref[...] + br_ref[...]
    safe = d > 0.0
    dn = jnp.where(safe, d, 1.0)
    agg = jnp.where(safe[:, None], (u + b * d[:, None]) / dn[:, None], 0.0)
    o_ref[...] = jnp.maximum(agg + hr, 0.0)


def _combine_stage(UD, h, WB, Wr, br):
    grid = _N // _BN
    nu = UD.shape[0]
    return pl.pallas_call(
        _combine_body,
        grid=(grid,),
        in_specs=[
            pl.BlockSpec((nu, _BN, 2 * _NHID), lambda i: (0, i, 0)),
            pl.BlockSpec((_BN, _NHID), lambda i: (i, 0)),
            pl.BlockSpec((_NHID, _NHID), lambda i: (0, 0)),
            pl.BlockSpec((_NHID, _NHID), lambda i: (0, 0)),
            pl.BlockSpec((1, _NHID), lambda i: (0, 0)),
        ],
        out_specs=pl.BlockSpec((_BN, _NHID), lambda i: (i, 0)),
        out_shape=jax.ShapeDtypeStruct((_N, _NHID), jnp.float32),
    )(UD, h, WB, Wr, br.reshape(1, _NHID))


def _proj_body(h_ref, w_ref, b_ref, o_ref):
    o_ref[...] = h_ref[...] @ w_ref[...] + b_ref[...]


def _proj_out(h, Wout, bout):
    grid = _N // _BN
    return pl.pallas_call(
        _proj_body,
        grid=(grid,),
        in_specs=[
            pl.BlockSpec((_BN, _NHID), lambda i: (i, 0)),
            pl.BlockSpec((_NHID, 2), lambda i: (0, 0)),
            pl.BlockSpec((1, 2), lambda i: (0, 0)),
        ],
        out_specs=pl.BlockSpec((_BN, 2), lambda i: (i, 0)),
        out_shape=jax.ShapeDtypeStruct((_N, 2), jnp.float32),
    )(h, Wout, bout.reshape(1, 2))


def _node_body(h_ref, wq_ref, bq_ref, wka_ref, wvv_ref, bvbe_ref,
               qt_ref, avt_ref):
    h = h_ref[...]
    qt_ref[...] = (h @ wq_ref[...] + bq_ref[...]) * 0.25
    avt_ref[...] = jnp.concatenate(
        [h @ wka_ref[...], h @ wvv_ref[...] + bvbe_ref[...]], axis=1)


def _node_stage(h, Wq, bq, WkA, WvV, bvbe):
    grid = _N // _BN
    return pl.pallas_call(
        _node_body,
        grid=(grid,),
        in_specs=[
            pl.BlockSpec((_BN, _NHID), lambda i: (i, 0)),
            pl.BlockSpec((_NHID, _NHID), lambda i: (0, 0)),
            pl.BlockSpec((1, _NHID), lambda i: (0, 0)),
            pl.BlockSpec((_NHID, _NHID), lambda i: (0, 0)),
            pl.BlockSpec((_NHID, _NHID), lambda i: (0, 0)),
            pl.BlockSpec((1, _NHID), lambda i: (0, 0)),
        ],
        out_specs=[
            pl.BlockSpec((_BN, _NHID), lambda i: (i, 0)),
            pl.BlockSpec((_BN, 2 * _NHID), lambda i: (i, 0)),
        ],
        out_shape=[
            jax.ShapeDtypeStruct((_N, _NHID), jnp.float32),
            jax.ShapeDtypeStruct((_N, 2 * _NHID), jnp.float32),
        ],
    )(h, Wq, bq.reshape(1, _NHID), WkA, WvV, bvbe.reshape(1, _NHID))


def _sc_mesh():
    return plsc.VectorSubcoreMesh(core_axis_name="c", subcore_axis_name="s")


@functools.cache
def _edge_sc_kernel():
    @functools.partial(
        pl.kernel,
        mesh=_sc_mesh(),
        out_type=jax.ShapeDtypeStruct((2, 16, _NP // 16, 2 * _NHID),
                                      jnp.float32),
        scratch_types=[
            pltpu.VMEM((_NCH, _KC), jnp.int32),
            pltpu.VMEM((_NCH, _KC), jnp.int32),
            pltpu.VMEM((2, _KC, _NHID), jnp.float32),
            pltpu.VMEM((2, _KC, 2 * _NHID), jnp.float32),
            pltpu.VMEM((2, _KC, 2 * _NHID), jnp.float32),
            pltpu.VMEM_SHARED((_NP, 2 * _NHID), jnp.float32),
            pltpu.SemaphoreType.DMA,
            pltpu.SemaphoreType.DMA,
            pltpu.SemaphoreType.DMA,
        ],
        compiler_params=pltpu.CompilerParams(use_tc_tiling_on_sc=False,
                                             needs_layout_passes=False),
    )
    def ek(qt_hbm, avt_hbm, srcr_hbm, dstr_hbm, zero_hbm, out_hbm,
           sidx, didx, qbuf, avbuf, mbuf, acc, semq, sema, semm):
        c = lax.axis_index("c")
        s = lax.axis_index("s")
        wid = s * 2 + c

        @pl.when(s == 0)
        def _():
            pltpu.sync_copy(zero_hbm, acc)

        plsc.subcore_barrier()
        pltpu.sync_copy(srcr_hbm.at[wid], sidx)
        pltpu.sync_copy(dstr_hbm.at[wid], didx)

        pltpu.async_copy(qt_hbm.at[didx.at[0]], qbuf.at[0], semq)
        pltpu.async_copy(avt_hbm.at[sidx.at[0]], avbuf.at[0], sema)

        def chunk(j, carry):
            sl = j & 1
            nxt = 1 - sl

            @pl.when(j + 1 < _NCH)
            def _():
                pltpu.async_copy(qt_hbm.at[didx.at[j + 1]], qbuf.at[nxt],
                                 semq)
                pltpu.async_copy(avt_hbm.at[sidx.at[j + 1]], avbuf.at[nxt],
                                 sema)

            pltpu.make_async_copy(qt_hbm.at[didx.at[j]], qbuf.at[sl],
                                  semq).wait()
            pltpu.make_async_copy(avt_hbm.at[sidx.at[j]], avbuf.at[sl],
                                  sema).wait()

            @plsc.parallel_loop(0, _KC, 1, unroll=8)
            def body(i):
                q = qbuf[sl, i, :]
                a = avbuf[sl, i, pl.ds(0, _NHID)]
                v = avbuf[sl, i, pl.ds(_NHID, _NHID)]
                logit = jnp.sum(q * a)
                p = jnp.exp(jnp.broadcast_to(jnp.minimum(logit, 60.0),
                                             (_NHID,)))
                mbuf[sl, i, pl.ds(0, _NHID)] = p * v
                mbuf[sl, i, pl.ds(_NHID, _NHID)] = p

            @pl.when(j > 0)
            def _():
                pltpu.make_async_copy(zero_hbm.at[pl.ds(0, _KC)],
                                      mbuf.at[nxt], semm).wait()

            pltpu.async_copy(mbuf.at[sl], acc.at[didx.at[j]], semm,
                             add=True)
            return carry

        lax.fori_loop(0, _NCH, chunk, 0)
        pltpu.make_async_copy(zero_hbm.at[pl.ds(0, _KC)], mbuf.at[0],
                              semm).wait()
        plsc.subcore_barrier()
        rows = _NP // 16
        pltpu.sync_copy(acc.at[pl.ds(s * rows, rows)], out_hbm.at[c, s])

    return ek


def kernel(x, pos, norm, W0, b0, W1, b1, Wq, bq, Wk, bk, Wv, bv, We, be,
           Wr, br, Wout, bout, edge_index):
    srcr = edge_index[0].reshape(_NW, _NCH, _KC)
    dstr = edge_index[1].reshape(_NW, _NCH, _KC)
    zero = jnp.zeros((_NP, 2 * _NHID), jnp.float32)
    ek = _edge_sc_kernel()
    h = _lin_in(x, W0, b0, W1, b1)
    for l in range(_DEPTH):
        WkA = Wk[l] + We[l][:_NHID]
        WvV = Wv[l] + We[l][:_NHID]
        bvbe = bv[l] + be[l]
        WB = We[l][_NHID:]
        QT, AVT = _node_stage(h, Wq[l], bq[l], WkA, WvV, bvbe)
        UD4 = ek(QT, AVT, srcr, dstr, zero)
        UD = UD4.reshape(2, _NP, 2 * _NHID)
        h = _combine_stage(UD, h, WB, Wr[l], br[l])
    return _proj_out(h, Wout, bout)


# speedup vs baseline: 68.5879x; 1.0865x over previous
import functools

import jax
import jax.numpy as jnp
from jax import lax
from jax.experimental import pallas as pl
from jax.experimental.pallas import tpu as pltpu
from jax.experimental.pallas import tpu_sc as plsc

_N = 10000
_E = 320000
_NHID = 16
_DEPTH = 3

_NW = 32
_KC = 100
_EPW = _E // _NW
_NCH = _EPW // _KC
_NP = 10240

_BN = 2000
_BE = 8000


def _lin_in_body(x_ref, w0_ref, b0_ref, w1_ref, b1_ref, o_ref):
    h = jnp.maximum(x_ref[...] @ w0_ref[...] + b0_ref[...], 0.0)
    o_ref[...] = jnp.maximum(h @ w1_ref[...] + b1_ref[...], 0.0)


def _lin_in(x, W0, b0, W1, b1):
    grid = _N // _BN
    return pl.pallas_call(
        _lin_in_body,
        grid=(grid,),
        in_specs=[
            pl.BlockSpec((_BN, 128), lambda i: (i, 0)),
            pl.BlockSpec((128, 128), lambda i: (0, 0)),
            pl.BlockSpec((1, 128), lambda i: (0, 0)),
            pl.BlockSpec((128, _NHID), lambda i: (0, 0)),
            pl.BlockSpec((1, _NHID), lambda i: (0, 0)),
        ],
        out_specs=pl.BlockSpec((_BN, _NHID), lambda i: (i, 0)),
        out_shape=jax.ShapeDtypeStruct((_N, _NHID), jnp.float32),
    )(x, W0, b0.reshape(1, 128), W1, b1.reshape(1, _NHID))


def _combine_body(ud_ref, h_ref, wb_ref, wr_ref, br_ref, o_ref):
    ud = jnp.sum(ud_ref[...], axis=0)
    u = ud[:, :_NHID]
    d = ud[:, _NHID]
    h = h_ref[...]
    b = h @ wb_ref[...]
    hr = h @ wr_ref[...] + br_ref[...]
    safe = d > 0.0
    dn = jnp.where(safe, d, 1.0)
    agg = jnp.where(safe[:, None], (u + b * d[:, None]) / dn[:, None], 0.0)
    o_ref[...] = jnp.maximum(agg + hr, 0.0)


def _combine_stage(UD, h, WB, Wr, br):
    grid = _N // _BN
    nu = UD.shape[0]
    return pl.pallas_call(
        _combine_body,
        grid=(grid,),
        in_specs=[
            pl.BlockSpec((nu, _BN, 2 * _NHID), lambda i: (0, i, 0)),
            pl.BlockSpec((_BN, _NHID), lambda i: (i, 0)),
            pl.BlockSpec((_NHID, _NHID), lambda i: (0, 0)),
            pl.BlockSpec((_NHID, _NHID), lambda i: (0, 0)),
            pl.BlockSpec((1, _NHID), lambda i: (0, 0)),
        ],
        out_specs=pl.BlockSpec((_BN, _NHID), lambda i: (i, 0)),
        out_shape=jax.ShapeDtypeStruct((_N, _NHID), jnp.float32),
    )(UD, h, WB, Wr, br.reshape(1, _NHID))


def _proj_body(h_ref, w_ref, b_ref, o_ref):
    o_ref[...] = h_ref[...] @ w_ref[...] + b_ref[...]


def _proj_out(h, Wout, bout):
    grid = _N // _BN
    return pl.pallas_call(
        _proj_body,
        grid=(grid,),
        in_specs=[
            pl.BlockSpec((_BN, _NHID), lambda i: (i, 0)),
            pl.BlockSpec((_NHID, 2), lambda i: (0, 0)),
            pl.BlockSpec((1, 2), lambda i: (0, 0)),
        ],
        out_specs=pl.BlockSpec((_BN, 2), lambda i: (i, 0)),
        out_shape=jax.ShapeDtypeStruct((_N, 2), jnp.float32),
    )(h, Wout, bout.reshape(1, 2))


def _node_body(h_ref, wq_ref, bq_ref, wka_ref, wvv_ref, bvbe_ref,
               qt_ref, avt_ref):
    h = h_ref[...]
    qt_ref[...] = (h @ wq_ref[...] + bq_ref[...]) * 0.25
    avt_ref[...] = jnp.concatenate(
        [h @ wka_ref[...], h @ wvv_ref[...] + bvbe_ref[...]], axis=1)


def _node_stage(h, Wq, bq, WkA, WvV, bvbe):
    grid = _N // _BN
    return pl.pallas_call(
        _node_body,
        grid=(grid,),
        in_specs=[
            pl.BlockSpec((_BN, _NHID), lambda i: (i, 0)),
            pl.BlockSpec((_NHID, _NHID), lambda i: (0, 0)),
            pl.BlockSpec((1, _NHID), lambda i: (0, 0)),
            pl.BlockSpec((_NHID, _NHID), lambda i: (0, 0)),
            pl.BlockSpec((_NHID, _NHID), lambda i: (0, 0)),
            pl.BlockSpec((1, _NHID), lambda i: (0, 0)),
        ],
        out_specs=[
            pl.BlockSpec((_BN, _NHID), lambda i: (i, 0)),
            pl.BlockSpec((_BN, 2 * _NHID), lambda i: (i, 0)),
        ],
        out_shape=[
            jax.ShapeDtypeStruct((_N, _NHID), jnp.float32),
            jax.ShapeDtypeStruct((_N, 2 * _NHID), jnp.float32),
        ],
    )(h, Wq, bq.reshape(1, _NHID), WkA, WvV, bvbe.reshape(1, _NHID))


def _sc_mesh():
    return plsc.VectorSubcoreMesh(core_axis_name="c", subcore_axis_name="s")


@functools.cache
def _edge_sc_kernel():
    @functools.partial(
        pl.kernel,
        mesh=_sc_mesh(),
        out_type=jax.ShapeDtypeStruct((2, 16, _NP // 16, 2 * _NHID),
                                      jnp.float32),
        scratch_types=[
            pltpu.VMEM((_NCH, _KC), jnp.int32),
            pltpu.VMEM((_NCH, _KC), jnp.int32),
            pltpu.VMEM((2, _KC, _NHID), jnp.float32),
            pltpu.VMEM((2, _KC, 2 * _NHID), jnp.float32),
            pltpu.VMEM((2, _KC, 2 * _NHID), jnp.float32),
            pltpu.VMEM_SHARED((_NP, 2 * _NHID), jnp.float32),
            pltpu.SemaphoreType.DMA,
            pltpu.SemaphoreType.DMA,
            pltpu.SemaphoreType.DMA,
        ],
        compiler_params=pltpu.CompilerParams(use_tc_tiling_on_sc=False,
                                             needs_layout_passes=False),
    )
    def ek(qt_hbm, avt_hbm, srcr_hbm, dstr_hbm, zero_hbm, out_hbm,
           sidx, didx, qbuf, avbuf, mbuf, acc, semq, sema, semm):
        c = lax.axis_index("c")
        s = lax.axis_index("s")
        wid = s * 2 + c

        @pl.when(s == 0)
        def _():
            pltpu.sync_copy(zero_hbm, acc)

        plsc.subcore_barrier()
        pltpu.sync_copy(srcr_hbm.at[wid], sidx)
        pltpu.sync_copy(dstr_hbm.at[wid], didx)

        pltpu.async_copy(qt_hbm.at[didx.at[0]], qbuf.at[0], semq)
        pltpu.async_copy(avt_hbm.at[sidx.at[0]], avbuf.at[0], sema)

        def chunk(j, carry):
            sl = j & 1
            nxt = 1 - sl

            @pl.when(j + 1 < _NCH)
            def _():
                pltpu.async_copy(qt_hbm.at[didx.at[j + 1]], qbuf.at[nxt],
                                 semq)
                pltpu.async_copy(avt_hbm.at[sidx.at[j + 1]], avbuf.at[nxt],
                                 sema)

            pltpu.make_async_copy(qt_hbm.at[didx.at[j]], qbuf.at[sl],
                                  semq).wait()
            pltpu.make_async_copy(avt_hbm.at[sidx.at[j]], avbuf.at[sl],
                                  sema).wait()

            lane15 = jnp.full((_NHID, 1), _NHID - 1, jnp.int32)
            dnums = lax.GatherDimensionNumbers(
                offset_dims=(), collapsed_slice_dims=(0,),
                start_index_map=(0,))

            @plsc.parallel_loop(0, _KC, 1, unroll=10)
            def body(i):
                q = qbuf[sl, i, :]
                a = avbuf[sl, i, pl.ds(0, _NHID)]
                v = avbuf[sl, i, pl.ds(_NHID, _NHID)]
                acc_l = lax.cumsum(q * a)
                logit = lax.gather(
                    acc_l, lane15, dnums, (1,),
                    mode=lax.GatherScatterMode.PROMISE_IN_BOUNDS)
                p = jnp.exp(jnp.minimum(logit, 60.0))
                mbuf[sl, i, pl.ds(0, _NHID)] = p * v
                mbuf[sl, i, pl.ds(_NHID, _NHID)] = p

            @pl.when(j > 0)
            def _():
                pltpu.make_async_copy(zero_hbm.at[pl.ds(0, _KC)],
                                      mbuf.at[nxt], semm).wait()

            pltpu.async_copy(mbuf.at[sl], acc.at[didx.at[j]], semm,
                             add=True)
            return carry

        lax.fori_loop(0, _NCH, chunk, 0)
        pltpu.make_async_copy(zero_hbm.at[pl.ds(0, _KC)], mbuf.at[0],
                              semm).wait()
        plsc.subcore_barrier()
        rows = _NP // 16
        pltpu.sync_copy(acc.at[pl.ds(s * rows, rows)], out_hbm.at[c, s])

    return ek


def kernel(x, pos, norm, W0, b0, W1, b1, Wq, bq, Wk, bk, Wv, bv, We, be,
           Wr, br, Wout, bout, edge_index):
    srcr = edge_index[0].reshape(_NW, _NCH, _KC)
    dstr = edge_index[1].reshape(_NW, _NCH, _KC)
    zero = jnp.zeros((_NP, 2 * _NHID), jnp.float32)
    ek = _edge_sc_kernel()
    h = _lin_in(x, W0, b0, W1, b1)
    for l in range(_DEPTH):
        WkA = Wk[l] + We[l][:_NHID]
        WvV = Wv[l] + We[l][:_NHID]
        bvbe = bv[l] + be[l]
        WB = We[l][_NHID:]
        QT, AVT = _node_stage(h, Wq[l], bq[l], WkA, WvV, bvbe)
        UD4 = ek(QT, AVT, srcr, dstr, zero)
        UD = UD4.reshape(2, _NP, 2 * _NHID)
        h = _combine_stage(UD, h, WB, Wr[l], br[l])
    return _proj_out(h, Wout, bout)
